# baseline (speedup 1.0000x reference)
import functools

import jax
import jax.numpy as jnp
from jax.experimental import pallas as pl
from jax.experimental.pallas import tpu as pltpu

B, C, N = 8, 3, 4096
MBLK = 1024


def _chamfer_block(q_ref, r_ref, out_ref):
    q = q_ref[0]
    r = r_ref[0]
    inner = jax.lax.dot_general(
        q, r, (((0,), (0,)), ((), ())), preferred_element_type=jnp.float32
    )
    sq_r = jnp.sum(r * r, axis=0)
    t = sq_r[None, :] - 2.0 * inner
    rowmin = jnp.min(t, axis=1)
    sq_q = jnp.sum(q * q, axis=0)
    d = jnp.maximum(rowmin + sq_q, 0.0)
    out_ref[pl.program_id(0), pl.program_id(1)] = jnp.sum(d)


@jax.jit
def kernel(pc2, pc1_warped):
    mb = N // MBLK
    partials = pl.pallas_call(
        _chamfer_block,
        grid=(B, mb),
        in_specs=[
            pl.BlockSpec((1, C, MBLK), lambda b, m: (b, 0, m)),
            pl.BlockSpec((1, C, N), lambda b, m: (b, 0, 0)),
        ],
        out_specs=pl.BlockSpec(memory_space=pltpu.SMEM),
        out_shape=jax.ShapeDtypeStruct((B, mb), jnp.float32),
    )(pc1_warped, pc2)
    return jnp.sum(partials) * (2.0 / B)


# speedup vs baseline: 1.3535x; 1.3535x over previous
import jax
import jax.numpy as jnp
from jax.experimental import pallas as pl
from jax.experimental.pallas import tpu as pltpu

B, C, N = 8, 3, 4096
MBLK = 1024
NCHUNK = 256


def _chamfer_block(q_ref, rt_ref, out_ref):
    q = q_ref[0]
    rt = rt_ref[0]
    qw = -2.0 * q
    sq_q = jnp.sum(q * q, axis=0)
    sq_r = jnp.sum(rt * rt, axis=1, keepdims=True)

    rowacc = jnp.full((MBLK,), jnp.inf, dtype=jnp.float32)
    for c in range(N // NCHUNK):
        sl = slice(c * NCHUNK, (c + 1) * NCHUNK)
        inner2 = jax.lax.dot_general(
            rt[sl, :], qw, (((1,), (0,)), ((), ())),
            preferred_element_type=jnp.float32,
        )
        t = sq_r[sl, :] + inner2
        rowacc = jnp.minimum(rowacc, jnp.min(t, axis=0))
    d = jnp.maximum(rowacc + sq_q, 0.0)
    out_ref[pl.program_id(0), pl.program_id(1)] = jnp.sum(d)


@jax.jit
def kernel(pc2, pc1_warped):
    rt = jnp.transpose(pc2, (0, 2, 1))
    mb = N // MBLK
    partials = pl.pallas_call(
        _chamfer_block,
        grid=(B, mb),
        in_specs=[
            pl.BlockSpec((1, C, MBLK), lambda b, m: (b, 0, m)),
            pl.BlockSpec((1, N, C), lambda b, m: (b, 0, 0)),
        ],
        out_specs=pl.BlockSpec(memory_space=pltpu.SMEM),
        out_shape=jax.ShapeDtypeStruct((B, mb), jnp.float32),
    )(pc1_warped, rt)
    return jnp.sum(partials) * (2.0 / B)
